# baseline (speedup 1.0000x reference)
import functools

import jax
import jax.numpy as jnp
from jax import lax
from jax.experimental import pallas as pl
from jax.experimental.pallas import tpu as pltpu
from jax.experimental.pallas import tpu_sc as plsc

B, C, H, W = 8, 96, 112, 112
K = 2
NH, NW_ = H * K, W * K
PLANES = B * C
LANES = 16

RIN = 56
ROUT = RIN * K
SPLITS = H // RIN
UNITS = PLANES * SPLITS
N_WORKERS = 32
PER_WORKER = UNITS // N_WORKERS

_mesh = plsc.VectorSubcoreMesh(core_axis_name="c", subcore_axis_name="s")


def _scatter_block(in_v, out_v):
    two_iota = lax.iota(jnp.int32, LANES) * 2

    def row_body(i, carry):
        row_idx = jnp.full((LANES,), 2 * i + 1, jnp.int32)
        for q in range(W // LANES):
            x = in_v[i, pl.ds(q * LANES, LANES)]
            col_idx = two_iota + (2 * q * LANES + 1)
            plsc.store_scatter(out_v, [row_idx, col_idx], x)
        return carry

    lax.fori_loop(0, RIN, row_body, 0)


@functools.partial(
    pl.kernel,
    out_type=jax.ShapeDtypeStruct((B, C, NH, NW_), jnp.float32),
    mesh=_mesh,
    scratch_types=[
        pltpu.VMEM((RIN, W), jnp.float32),
        pltpu.VMEM((RIN, W), jnp.float32),
        pltpu.VMEM((ROUT, NW_), jnp.float32),
        pltpu.VMEM((ROUT, NW_), jnp.float32),
        pltpu.SemaphoreType.DMA,
        pltpu.SemaphoreType.DMA,
        pltpu.SemaphoreType.DMA,
        pltpu.SemaphoreType.DMA,
    ],
    compiler_params=pltpu.CompilerParams(
        needs_layout_passes=False, use_tc_tiling_on_sc=True),
)
def _unpool_sc(in_hbm, zero_hbm, out_hbm, in_v0, in_v1, out_v0, out_v1,
               sem_i0, sem_i1, sem_o0, sem_o1):
    wid = lax.axis_index("s") * 2 + lax.axis_index("c")
    base_unit = wid * PER_WORKER

    in_v = [in_v0, in_v1]
    out_v = [out_v0, out_v1]
    sem_i = [sem_i0, sem_i1]
    sem_o = [sem_o0, sem_o1]

    def in_slice(u):
        unit = base_unit + u
        plane = unit // SPLITS
        half = unit % SPLITS
        return in_hbm.at[plane // C, plane % C, pl.ds(half * RIN, RIN)]

    def out_slice(u):
        unit = base_unit + u
        plane = unit // SPLITS
        half = unit % SPLITS
        return out_hbm.at[plane // C, plane % C, pl.ds(half * ROUT, ROUT)]

    in_descs = [None, None]
    out_descs = [
        pltpu.async_copy(zero_hbm, out_v[0], sem_o[0]),
        pltpu.async_copy(zero_hbm, out_v[1], sem_o[1]),
    ]
    in_descs[0] = pltpu.async_copy(in_slice(0), in_v[0], sem_i[0])
    for p in range(PER_WORKER):
        u = p % 2
        if p + 1 < PER_WORKER:
            nu = (p + 1) % 2
            in_descs[nu] = pltpu.async_copy(in_slice(p + 1), in_v[nu], sem_i[nu])
        in_descs[u].wait()
        out_descs[u].wait()
        _scatter_block(in_v[u], out_v[u])
        out_descs[u] = pltpu.async_copy(out_v[u], out_slice(p), sem_o[u])
    out_descs[(PER_WORKER - 2) % 2].wait()
    out_descs[(PER_WORKER - 1) % 2].wait()


def kernel(input):
    zeros = jnp.zeros((ROUT, NW_), jnp.float32)
    return _unpool_sc(input, zeros)


# speedup vs baseline: 1.0643x; 1.0643x over previous
import functools

import jax
import jax.numpy as jnp
from jax import lax
from jax.experimental import pallas as pl
from jax.experimental.pallas import tpu as pltpu
from jax.experimental.pallas import tpu_sc as plsc

B, C, H, W = 8, 96, 112, 112
K = 2
NH, NW_ = H * K, W * K
PLANES = B * C
LANES = 16

RIN = 56
ROUT = RIN * K
SPLITS = H // RIN
UNITS = PLANES * SPLITS
N_WORKERS = 32
PER_WORKER = UNITS // N_WORKERS

_mesh = plsc.VectorSubcoreMesh(core_axis_name="c", subcore_axis_name="s")


def _zero_fill(out_v):
    zero = jnp.zeros((LANES,), jnp.float32)

    def zrow(r, carry):
        for t in range(NW_ // LANES):
            out_v[r, pl.ds(t * LANES, LANES)] = zero
        return carry

    lax.fori_loop(0, ROUT, zrow, 0)


def _scatter_block(in_v, out_v):
    two_iota = lax.iota(jnp.int32, LANES) * 2

    def row_body(i, carry):
        row_idx = jnp.full((LANES,), 2 * i + 1, jnp.int32)
        for q in range(W // LANES):
            x = in_v[i, pl.ds(q * LANES, LANES)]
            col_idx = two_iota + (2 * q * LANES + 1)
            plsc.store_scatter(out_v, [row_idx, col_idx], x)
        return carry

    lax.fori_loop(0, RIN, row_body, 0)


@functools.partial(
    pl.kernel,
    out_type=jax.ShapeDtypeStruct((B, C, NH, NW_), jnp.float32),
    mesh=_mesh,
    scratch_types=[
        pltpu.VMEM((RIN, W), jnp.float32),
        pltpu.VMEM((RIN, W), jnp.float32),
        pltpu.VMEM((ROUT, NW_), jnp.float32),
        pltpu.VMEM((ROUT, NW_), jnp.float32),
        pltpu.SemaphoreType.DMA,
        pltpu.SemaphoreType.DMA,
        pltpu.SemaphoreType.DMA,
        pltpu.SemaphoreType.DMA,
    ],
    compiler_params=pltpu.CompilerParams(
        needs_layout_passes=False, use_tc_tiling_on_sc=True),
)
def _unpool_sc(in_hbm, out_hbm, in_v0, in_v1, out_v0, out_v1,
               sem_i0, sem_i1, sem_o0, sem_o1):
    wid = lax.axis_index("s") * 2 + lax.axis_index("c")
    base_unit = wid * PER_WORKER

    in_v = [in_v0, in_v1]
    out_v = [out_v0, out_v1]
    sem_i = [sem_i0, sem_i1]
    sem_o = [sem_o0, sem_o1]

    def in_slice(u):
        unit = base_unit + u
        plane = unit // SPLITS
        half = unit % SPLITS
        return in_hbm.at[plane // C, plane % C, pl.ds(half * RIN, RIN)]

    def out_slice(u):
        unit = base_unit + u
        plane = unit // SPLITS
        half = unit % SPLITS
        return out_hbm.at[plane // C, plane % C, pl.ds(half * ROUT, ROUT)]

    in_descs = [None, None]
    out_descs = [None, None]
    in_descs[0] = pltpu.async_copy(in_slice(0), in_v[0], sem_i[0])
    in_descs[1] = pltpu.async_copy(in_slice(1), in_v[1], sem_i[1])
    _zero_fill(out_v[0])
    in_descs[0].wait()
    _scatter_block(in_v[0], out_v[0])
    out_descs[0] = pltpu.async_copy(out_v[0], out_slice(0), sem_o[0])
    _zero_fill(out_v[1])
    for p in range(1, PER_WORKER):
        u = p % 2
        if p + 1 < PER_WORKER:
            nu = (p + 1) % 2
            in_descs[nu] = pltpu.async_copy(in_slice(p + 1), in_v[nu], sem_i[nu])
        in_descs[u].wait()
        if p >= 2:
            out_descs[u].wait()
        _scatter_block(in_v[u], out_v[u])
        out_descs[u] = pltpu.async_copy(out_v[u], out_slice(p), sem_o[u])
    out_descs[(PER_WORKER - 2) % 2].wait()
    out_descs[(PER_WORKER - 1) % 2].wait()


def kernel(input):
    return _unpool_sc(input)


# speedup vs baseline: 1.0697x; 1.0051x over previous
import functools

import jax
import jax.numpy as jnp
from jax import lax
from jax.experimental import pallas as pl
from jax.experimental.pallas import tpu as pltpu
from jax.experimental.pallas import tpu_sc as plsc

B, C, H, W = 8, 96, 112, 112
K = 2
NH, NW_ = H * K, W * K
PLANES = B * C
LANES = 16

RIN = 56
ROUT = RIN * K
SPLITS = H // RIN
UNITS = PLANES * SPLITS
N_WORKERS = 32
PER_WORKER = UNITS // N_WORKERS

_mesh = plsc.VectorSubcoreMesh(core_axis_name="c", subcore_axis_name="s")


def _zero_fill(out_v):
    zero = jnp.zeros((LANES,), jnp.float32)

    @plsc.parallel_loop(0, ROUT)
    def _(r):
        for t in range(NW_ // LANES):
            out_v[r, pl.ds(t * LANES, LANES)] = zero


def _scatter_block(in_v, out_v):
    two_iota = lax.iota(jnp.int32, LANES) * 2

    @plsc.parallel_loop(0, RIN, unroll=2)
    def _(i):
        row_idx = jnp.full((LANES,), 2 * i + 1, jnp.int32)
        for q in range(W // LANES):
            x = in_v[i, pl.ds(q * LANES, LANES)]
            col_idx = two_iota + (2 * q * LANES + 1)
            plsc.store_scatter(out_v, [row_idx, col_idx], x)


@functools.partial(
    pl.kernel,
    out_type=jax.ShapeDtypeStruct((B, C, NH, NW_), jnp.float32),
    mesh=_mesh,
    scratch_types=[
        pltpu.VMEM((RIN, W), jnp.float32),
        pltpu.VMEM((RIN, W), jnp.float32),
        pltpu.VMEM((ROUT, NW_), jnp.float32),
        pltpu.VMEM((ROUT, NW_), jnp.float32),
        pltpu.SemaphoreType.DMA,
        pltpu.SemaphoreType.DMA,
        pltpu.SemaphoreType.DMA,
        pltpu.SemaphoreType.DMA,
    ],
    compiler_params=pltpu.CompilerParams(
        needs_layout_passes=False, use_tc_tiling_on_sc=True),
)
def _unpool_sc(in_hbm, out_hbm, in_v0, in_v1, out_v0, out_v1,
               sem_i0, sem_i1, sem_o0, sem_o1):
    wid = lax.axis_index("s") * 2 + lax.axis_index("c")
    base_unit = wid * PER_WORKER

    in_v = [in_v0, in_v1]
    out_v = [out_v0, out_v1]
    sem_i = [sem_i0, sem_i1]
    sem_o = [sem_o0, sem_o1]

    def in_slice(u):
        unit = base_unit + u
        plane = unit // SPLITS
        half = unit % SPLITS
        return in_hbm.at[plane // C, plane % C, pl.ds(half * RIN, RIN)]

    def out_slice(u):
        unit = base_unit + u
        plane = unit // SPLITS
        half = unit % SPLITS
        return out_hbm.at[plane // C, plane % C, pl.ds(half * ROUT, ROUT)]

    in_descs = [None, None]
    out_descs = [None, None]
    in_descs[0] = pltpu.async_copy(in_slice(0), in_v[0], sem_i[0])
    in_descs[1] = pltpu.async_copy(in_slice(1), in_v[1], sem_i[1])
    _zero_fill(out_v[0])
    in_descs[0].wait()
    _scatter_block(in_v[0], out_v[0])
    out_descs[0] = pltpu.async_copy(out_v[0], out_slice(0), sem_o[0])
    _zero_fill(out_v[1])
    for p in range(1, PER_WORKER):
        u = p % 2
        if p + 1 < PER_WORKER:
            nu = (p + 1) % 2
            in_descs[nu] = pltpu.async_copy(in_slice(p + 1), in_v[nu], sem_i[nu])
        in_descs[u].wait()
        if p >= 2:
            out_descs[u].wait()
        _scatter_block(in_v[u], out_v[u])
        out_descs[u] = pltpu.async_copy(out_v[u], out_slice(p), sem_o[u])
    out_descs[(PER_WORKER - 2) % 2].wait()
    out_descs[(PER_WORKER - 1) % 2].wait()


def kernel(input):
    return _unpool_sc(input)
